# baseline (speedup 1.0000x reference)
import functools

import jax
import jax.numpy as jnp
import numpy as np
from jax import lax
from jax.experimental import pallas as pl
from jax.experimental.pallas import tpu as pltpu
from jax.experimental.pallas import tpu_sc as plsc

N = 1024
E = N * (N - 1) // 2
NC, NS, L = 2, 16, 16
NW = NC * NS
E_PER_W = E // NW
SPAN = 2 * E_PER_W
AT_PAD = 1056

NBLOCKS = E // 128
_ROW_COST = 60
_tri = (np.arange(N, dtype=np.int64) * (np.arange(N, dtype=np.int64) - 1)) // 2
_UNIT = 2048
_u_end = np.minimum(np.arange(1, (E + _UNIT - 1) // _UNIT + 1) * _UNIT, E)
_cost = _u_end + _ROW_COST * np.searchsorted(_tri, _u_end, side="left")
_targets = _cost[-1] * (np.arange(1, NW + 1) / NW)
_cuts = np.searchsorted(_cost, _targets, side="left")
_cuts[-1] = len(_u_end) - 1
_cuts = np.maximum.accumulate(np.minimum(_cuts, len(_u_end) - 1))
for _w in range(1, NW):
    if _cuts[_w] <= _cuts[_w - 1]:
        _cuts[_w] = _cuts[_w - 1] + 1
_ends = _u_end[_cuts]
_starts = np.concatenate([[0], _ends[:-1]])
_CNT = (_ends - _starts).astype(np.int32)
_BS = (_starts // 128).astype(np.int32)
MAX_BLK = int(np.max((_CNT + 127) // 128))
_ROW0 = (np.searchsorted(_tri, _starts, side="right") - 1).astype(np.int32)
_COL0 = (_starts - _tri[_ROW0]).astype(np.int32)


def _stage1_body(x_ref, cen_ref, w1s_ref, w1n_ref, b1_ref, w2s_ref, w2n_ref,
                 b2_ref, wm1_ref, bm1_ref, at_ref, b_ref):
    f32 = jnp.float32
    x = x_ref[...]
    cen = cen_ref[...]
    nrm = jnp.sqrt(jnp.sum(x * x, axis=1, keepdims=True))
    fn = x / jnp.maximum(nrm, 1e-12)
    h = jnp.concatenate([x, cen], axis=1)

    rj = lax.broadcasted_iota(jnp.int32, (N, N), 0)
    ci = lax.broadcasted_iota(jnp.int32, (N, N), 1)
    U = (ci > rj).astype(f32)

    g = x * fn
    agg32 = fn * jnp.dot(U, g, preferred_element_type=f32)

    cenT = cen.T
    parts = []
    for kk in range(3):
        cj = cen[:, kk:kk + 1]
        cirow = cenT[kk:kk + 1, :]
        w = jnp.abs(cj - cirow) * cirow * U
        parts.append(jnp.sum(w, axis=1, keepdims=True))
    agg3 = jnp.concatenate(parts, axis=1)

    agg = jnp.concatenate([agg32, agg3], axis=1)
    deg = (N - 1.0) - lax.broadcasted_iota(jnp.int32, (N, 1), 0).astype(f32)
    invdeg = 1.0 / jnp.maximum(deg, 1.0)
    hn1 = agg * invdeg
    h1 = (jnp.dot(h, w1s_ref[...], preferred_element_type=f32)
          + jnp.dot(hn1, w1n_ref[...], preferred_element_type=f32)
          + b1_ref[...][None, :])

    agg2 = jnp.dot(U, h1, preferred_element_type=f32)
    hn2 = agg2 * invdeg
    h2 = (jnp.dot(h1, w2s_ref[...], preferred_element_type=f32)
          + jnp.dot(hn2, w2n_ref[...], preferred_element_type=f32)
          + b2_ref[...][None, :])

    wm1 = wm1_ref[...]
    a_mat = jnp.dot(h2, wm1[:32, :], preferred_element_type=f32) + bm1_ref[...][None, :]
    b_mat = jnp.dot(h2, wm1[32:, :], preferred_element_type=f32)
    at_ref[:, :N] = a_mat.T
    at_ref[:, N:] = jnp.zeros((32, AT_PAD - N), f32)
    b_ref[...] = b_mat.T


def _stage2_body(at_hbm, b_hbm, wv_hbm, row0_hbm, col0_hbm,
                 cnt_hbm, bs_hbm, out_hbm,
                 at_v, b_v, wv_v, row0_v, col0_v, cnt_v, bs_v, out_v):
    i32 = jnp.int32
    wid = lax.axis_index("s") * NC + lax.axis_index("c")
    pltpu.sync_copy(at_hbm, at_v)
    pltpu.sync_copy(b_hbm, b_v)
    pltpu.sync_copy(wv_hbm, wv_v)
    pltpu.sync_copy(row0_hbm, row0_v)
    pltpu.sync_copy(col0_hbm, col0_v)
    pltpu.sync_copy(cnt_hbm, cnt_v)
    pltpu.sync_copy(bs_hbm, bs_v)

    def splat(v):
        return jnp.full((L,), v, i32)

    widv = splat(wid)
    i0 = jnp.max(plsc.load_gather(row0_v, [widv]))
    j0 = jnp.max(plsc.load_gather(col0_v, [widv]))
    cnt = jnp.max(plsc.load_gather(cnt_v, [widv]))
    bstart = jnp.max(plsc.load_gather(bs_v, [widv]))

    bm0 = plsc.load_gather(wv_v, [splat(2), splat(1)])
    bm1v = plsc.load_gather(wv_v, [splat(2), splat(2)])
    lane = lax.iota(i32, L)

    for half in range(2):
        ks = list(range(16 * half, 16 * half + 16))
        w0h = [plsc.load_gather(wv_v, [splat(0), splat(k + 1)]) for k in ks]
        w1h = [plsc.load_gather(wv_v, [splat(1), splat(k + 1)]) for k in ks]

        def row_body(state, half=half, ks=ks, w0h=w0h, w1h=w1h):
            i, jcur, ec = state
            seg = jnp.minimum(i - jcur, cnt - ec)
            iv = splat(i)
            bk = [plsc.load_gather(b_v, [splat(k), iv]) for k in ks]

            lead = jcur & (L - 1)
            base = jcur - lead
            nch = (lead + seg + (L - 1)) // L

            @plsc.parallel_loop(0, nch, unroll=2)
            def ch_body(c):
                off = base + c * L
                jj = off + lane
                m = (jj >= jcur) & (jj - jcur < seg)
                l = jnp.maximum(ec + jj - jcur, 0)
                idx0 = ((l >> 7) << 8) + (l & 127)
                if half == 0:
                    acc0 = bm0
                    acc1 = bm1v
                else:
                    acc0 = plsc.load_gather(out_v, [idx0])
                    acc1 = plsc.load_gather(out_v, [idx0 + 128])
                for kk, k in enumerate(ks):
                    a = at_v[k, pl.ds(off, L)]
                    t = jnp.maximum(a + bk[kk], 0.0)
                    acc0 = acc0 + t * w0h[kk]
                    acc1 = acc1 + t * w1h[kk]
                if half == 1:
                    acc0 = jnp.maximum(acc0, 0.0)
                    acc1 = jnp.maximum(acc1, 0.0)
                plsc.store_scatter(out_v, [idx0], acc0, mask=m)
                plsc.store_scatter(out_v, [idx0 + 128], acc1, mask=m)

            jn = jcur + seg
            done_row = jn >= i
            return (jnp.where(done_row, i + 1, i),
                    jnp.where(done_row, 0, jn),
                    ec + seg)

        lax.while_loop(lambda s: s[2] < cnt, row_body,
                       (i0, j0, jnp.int32(0)))
    base = bstart * 256
    nfull = cnt >> 11

    def dma_body(c, carry):
        pltpu.sync_copy(out_v.at[pl.ds(c * 4096, 4096)],
                        out_hbm.at[pl.ds(base + c * 4096, 4096)])
        return carry

    lax.fori_loop(0, nfull, dma_body, jnp.int32(0))

    @pl.when((cnt & 2047) != 0)
    def _():
        pltpu.sync_copy(out_v.at[pl.ds(nfull * 4096, 3072)],
                        out_hbm.at[pl.ds(base + nfull * 4096, 3072)])


@jax.jit
def kernel(x, centroids, W1_self, W1_neigh, b1, W2_self, W2_neigh, b2,
           Wm1, bm1, Wm2, bm2):
    f32 = jnp.float32
    at, b_mat = pl.pallas_call(
        _stage1_body,
        out_shape=[jax.ShapeDtypeStruct((32, AT_PAD), f32),
                   jax.ShapeDtypeStruct((32, N), f32)],
    )(x, centroids, W1_self, W1_neigh, b1, W2_self, W2_neigh, b2, Wm1, bm1)

    z1 = jnp.zeros((1,), f32)
    wv = jnp.stack([
        jnp.concatenate([z1, Wm2[:, 0], z1]),
        jnp.concatenate([z1, Wm2[:, 1], z1]),
        jnp.concatenate([z1, bm2, jnp.zeros((31,), f32)]),
        jnp.zeros((34,), f32),
    ])
    mesh = plsc.VectorSubcoreMesh(core_axis_name="c", subcore_axis_name="s")
    stage2 = functools.partial(
        pl.kernel,
        out_type=jax.ShapeDtypeStruct((2 * E,), f32),
        mesh=mesh,
        compiler_params=pltpu.CompilerParams(needs_layout_passes=False),
        scratch_types=[
            pltpu.VMEM((32, AT_PAD), f32),
            pltpu.VMEM((32, N), f32),
            pltpu.VMEM((4, 34), f32),
            pltpu.VMEM((NW,), jnp.int32),
            pltpu.VMEM((NW,), jnp.int32),
            pltpu.VMEM((NW,), jnp.int32),
            pltpu.VMEM((NW,), jnp.int32),
            pltpu.VMEM((MAX_BLK * 256,), f32),
        ],
    )(_stage2_body)
    flat = stage2(at, b_mat, wv, jnp.asarray(_ROW0), jnp.asarray(_COL0),
                  jnp.asarray(_CNT), jnp.asarray(_BS))
    return flat.reshape(NBLOCKS, 2, 128).transpose(0, 2, 1).reshape(E, 2)


# speedup vs baseline: 1.1346x; 1.1346x over previous
import functools

import jax
import jax.numpy as jnp
import numpy as np
from jax import lax
from jax.experimental import pallas as pl
from jax.experimental.pallas import tpu as pltpu
from jax.experimental.pallas import tpu_sc as plsc

N = 1024
E = N * (N - 1) // 2
NC, NS, L = 2, 16, 16
NW = NC * NS
E_PER_W = E // NW
SPAN = 2 * E_PER_W
AT_PAD = 1056

NBLOCKS = E // 128
_ROW_COST = 60
_tri = (np.arange(N, dtype=np.int64) * (np.arange(N, dtype=np.int64) - 1)) // 2
_UNIT = 2048
_u_end = np.minimum(np.arange(1, (E + _UNIT - 1) // _UNIT + 1) * _UNIT, E)
_cost = _u_end + _ROW_COST * np.searchsorted(_tri, _u_end, side="left")
_targets = _cost[-1] * (np.arange(1, NW + 1) / NW)
_cuts = np.searchsorted(_cost, _targets, side="left")
_cuts[-1] = len(_u_end) - 1
_cuts = np.maximum.accumulate(np.minimum(_cuts, len(_u_end) - 1))
for _w in range(1, NW):
    if _cuts[_w] <= _cuts[_w - 1]:
        _cuts[_w] = _cuts[_w - 1] + 1
_ends = _u_end[_cuts]
_starts = np.concatenate([[0], _ends[:-1]])
_CNT = (_ends - _starts).astype(np.int32)
_BS = (_starts // 128).astype(np.int32)
MAX_BLK = int(np.max((_CNT + 127) // 128))
_ROW0 = (np.searchsorted(_tri, _starts, side="right") - 1).astype(np.int32)
_COL0 = (_starts - _tri[_ROW0]).astype(np.int32)


def _stage1_body(x_ref, cen_ref, w1s_ref, w1n_ref, b1_ref, w2s_ref, w2n_ref,
                 b2_ref, wm1_ref, bm1_ref, at_ref, b_ref):
    f32 = jnp.float32
    x = x_ref[...]
    cen = cen_ref[...]
    nrm = jnp.sqrt(jnp.sum(x * x, axis=1, keepdims=True))
    fn = x / jnp.maximum(nrm, 1e-12)
    h = jnp.concatenate([x, cen], axis=1)

    rj = lax.broadcasted_iota(jnp.int32, (N, N), 0)
    ci = lax.broadcasted_iota(jnp.int32, (N, N), 1)
    U = (ci > rj).astype(f32)

    g = x * fn
    agg32 = fn * jnp.dot(U, g, preferred_element_type=f32)

    cenT = cen.T
    parts = []
    for kk in range(3):
        cj = cen[:, kk:kk + 1]
        cirow = cenT[kk:kk + 1, :]
        w = jnp.abs(cj - cirow) * cirow * U
        parts.append(jnp.sum(w, axis=1, keepdims=True))
    agg3 = jnp.concatenate(parts, axis=1)

    agg = jnp.concatenate([agg32, agg3], axis=1)
    deg = (N - 1.0) - lax.broadcasted_iota(jnp.int32, (N, 1), 0).astype(f32)
    invdeg = 1.0 / jnp.maximum(deg, 1.0)
    hn1 = agg * invdeg
    h1 = (jnp.dot(h, w1s_ref[...], preferred_element_type=f32)
          + jnp.dot(hn1, w1n_ref[...], preferred_element_type=f32)
          + b1_ref[...][None, :])

    agg2 = jnp.dot(U, h1, preferred_element_type=f32)
    hn2 = agg2 * invdeg
    h2 = (jnp.dot(h1, w2s_ref[...], preferred_element_type=f32)
          + jnp.dot(hn2, w2n_ref[...], preferred_element_type=f32)
          + b2_ref[...][None, :])

    wm1 = wm1_ref[...]
    a_mat = jnp.dot(h2, wm1[:32, :], preferred_element_type=f32) + bm1_ref[...][None, :]
    b_mat = jnp.dot(h2, wm1[32:, :], preferred_element_type=f32)
    at_ref[:, :N] = a_mat.T
    at_ref[:, N:] = jnp.zeros((32, AT_PAD - N), f32)
    b_ref[...] = b_mat.T


def _stage2_body(at_hbm, b_hbm, wv_hbm, row0_hbm, col0_hbm,
                 cnt_hbm, bs_hbm, out_hbm,
                 at_v, b_v, wv_v, row0_v, col0_v, cnt_v, bs_v, out_v):
    i32 = jnp.int32
    wid = lax.axis_index("s") * NC + lax.axis_index("c")
    pltpu.sync_copy(at_hbm, at_v)
    pltpu.sync_copy(b_hbm, b_v)
    pltpu.sync_copy(wv_hbm, wv_v)
    pltpu.sync_copy(row0_hbm, row0_v)
    pltpu.sync_copy(col0_hbm, col0_v)
    pltpu.sync_copy(cnt_hbm, cnt_v)
    pltpu.sync_copy(bs_hbm, bs_v)

    def splat(v):
        return jnp.full((L,), v, i32)

    widv = splat(wid)
    i0 = jnp.max(plsc.load_gather(row0_v, [widv]))
    j0 = jnp.max(plsc.load_gather(col0_v, [widv]))
    cnt = jnp.max(plsc.load_gather(cnt_v, [widv]))
    bstart = jnp.max(plsc.load_gather(bs_v, [widv]))

    bm0 = plsc.load_gather(wv_v, [splat(2), splat(1)])
    bm1v = plsc.load_gather(wv_v, [splat(2), splat(2)])
    lane = lax.iota(i32, L)

    for half in range(2):
        ks = list(range(16 * half, 16 * half + 16))
        w0h = [plsc.load_gather(wv_v, [splat(0), splat(k + 1)]) for k in ks]
        w1h = [plsc.load_gather(wv_v, [splat(1), splat(k + 1)]) for k in ks]

        def row_body(state, half=half, ks=ks, w0h=w0h, w1h=w1h):
            i, jcur, ec = state
            seg = jnp.minimum(i - jcur, cnt - ec)
            iv = splat(i)
            bk = [plsc.load_gather(b_v, [splat(k), iv]) for k in ks]

            lead = jcur & (L - 1)
            base = jcur - lead
            nch = (lead + seg + (L - 1)) // L

            def process_chunk(c):
                off = base + c * L
                jj = off + lane
                m = (jj >= jcur) & (jj - jcur < seg)
                l = jnp.maximum(ec + jj - jcur, 0)
                idx0 = ((l >> 7) << 8) + (l & 127)
                if half == 0:
                    acc0 = bm0
                    acc1 = bm1v
                else:
                    acc0 = plsc.load_gather(out_v, [idx0])
                    acc1 = plsc.load_gather(out_v, [idx0 + 128])
                for kk, k in enumerate(ks):
                    a = at_v[k, pl.ds(off, L)]
                    t = jnp.maximum(a + bk[kk], 0.0)
                    acc0 = acc0 + t * w0h[kk]
                    acc1 = acc1 + t * w1h[kk]
                if half == 1:
                    acc0 = jnp.maximum(acc0, 0.0)
                    acc1 = jnp.maximum(acc1, 0.0)
                plsc.store_scatter(out_v, [idx0], acc0, mask=m)
                plsc.store_scatter(out_v, [idx0 + 128], acc1, mask=m)

            def ch_pair(c, carry):
                process_chunk(2 * c)
                process_chunk(2 * c + 1)
                return carry

            lax.fori_loop(0, nch >> 1, ch_pair, jnp.int32(0))

            @pl.when((nch & 1) != 0)
            def _():
                process_chunk(nch - 1)

            jn = jcur + seg
            done_row = jn >= i
            return (jnp.where(done_row, i + 1, i),
                    jnp.where(done_row, 0, jn),
                    ec + seg)

        lax.while_loop(lambda s: s[2] < cnt, row_body,
                       (i0, j0, jnp.int32(0)))
    base = bstart * 256
    nfull = cnt >> 11

    def dma_body(c, carry):
        pltpu.sync_copy(out_v.at[pl.ds(c * 4096, 4096)],
                        out_hbm.at[pl.ds(base + c * 4096, 4096)])
        return carry

    lax.fori_loop(0, nfull, dma_body, jnp.int32(0))

    @pl.when((cnt & 2047) != 0)
    def _():
        pltpu.sync_copy(out_v.at[pl.ds(nfull * 4096, 3072)],
                        out_hbm.at[pl.ds(base + nfull * 4096, 3072)])


@jax.jit
def kernel(x, centroids, W1_self, W1_neigh, b1, W2_self, W2_neigh, b2,
           Wm1, bm1, Wm2, bm2):
    f32 = jnp.float32
    at, b_mat = pl.pallas_call(
        _stage1_body,
        out_shape=[jax.ShapeDtypeStruct((32, AT_PAD), f32),
                   jax.ShapeDtypeStruct((32, N), f32)],
    )(x, centroids, W1_self, W1_neigh, b1, W2_self, W2_neigh, b2, Wm1, bm1)

    z1 = jnp.zeros((1,), f32)
    wv = jnp.stack([
        jnp.concatenate([z1, Wm2[:, 0], z1]),
        jnp.concatenate([z1, Wm2[:, 1], z1]),
        jnp.concatenate([z1, bm2, jnp.zeros((31,), f32)]),
        jnp.zeros((34,), f32),
    ])
    mesh = plsc.VectorSubcoreMesh(core_axis_name="c", subcore_axis_name="s")
    stage2 = functools.partial(
        pl.kernel,
        out_type=jax.ShapeDtypeStruct((2 * E,), f32),
        mesh=mesh,
        compiler_params=pltpu.CompilerParams(needs_layout_passes=False),
        scratch_types=[
            pltpu.VMEM((32, AT_PAD), f32),
            pltpu.VMEM((32, N), f32),
            pltpu.VMEM((4, 34), f32),
            pltpu.VMEM((NW,), jnp.int32),
            pltpu.VMEM((NW,), jnp.int32),
            pltpu.VMEM((NW,), jnp.int32),
            pltpu.VMEM((NW,), jnp.int32),
            pltpu.VMEM((MAX_BLK * 256,), f32),
        ],
    )(_stage2_body)
    flat = stage2(at, b_mat, wv, jnp.asarray(_ROW0), jnp.asarray(_COL0),
                  jnp.asarray(_CNT), jnp.asarray(_BS))
    return flat.reshape(NBLOCKS, 2, 128).transpose(0, 2, 1).reshape(E, 2)


# speedup vs baseline: 1.2277x; 1.0820x over previous
import functools

import jax
import jax.numpy as jnp
import numpy as np
from jax import lax
from jax.experimental import pallas as pl
from jax.experimental.pallas import tpu as pltpu
from jax.experimental.pallas import tpu_sc as plsc

N = 1024
E = N * (N - 1) // 2
NC, NS, L = 2, 16, 16
NW = NC * NS
E_PER_W = E // NW
SPAN = 2 * E_PER_W
AT_PAD = 1056

NBLOCKS = E // 128
_ROW_COST = 30
_tri = (np.arange(N, dtype=np.int64) * (np.arange(N, dtype=np.int64) - 1)) // 2
_UNIT = 2048
_u_end = np.minimum(np.arange(1, (E + _UNIT - 1) // _UNIT + 1) * _UNIT, E)
_cost = _u_end + _ROW_COST * np.searchsorted(_tri, _u_end, side="left")
_targets = _cost[-1] * (np.arange(1, NW + 1) / NW)
_cuts = np.searchsorted(_cost, _targets, side="left")
_cuts[-1] = len(_u_end) - 1
_cuts = np.maximum.accumulate(np.minimum(_cuts, len(_u_end) - 1))
for _w in range(1, NW):
    if _cuts[_w] <= _cuts[_w - 1]:
        _cuts[_w] = _cuts[_w - 1] + 1
_ends = _u_end[_cuts]
_starts = np.concatenate([[0], _ends[:-1]])
_CNT = (_ends - _starts).astype(np.int32)
_BS = (_starts // 128).astype(np.int32)
MAX_BLK = int(np.max((_CNT + 127) // 128))
_ROW0 = (np.searchsorted(_tri, _starts, side="right") - 1).astype(np.int32)
_COL0 = (_starts - _tri[_ROW0]).astype(np.int32)


def _stage1_body(x_ref, cen_ref, w1s_ref, w1n_ref, b1_ref, w2s_ref, w2n_ref,
                 b2_ref, wm1_ref, bm1_ref, at_ref, b_ref):
    f32 = jnp.float32
    x = x_ref[...]
    cen = cen_ref[...]
    nrm = jnp.sqrt(jnp.sum(x * x, axis=1, keepdims=True))
    fn = x / jnp.maximum(nrm, 1e-12)
    h = jnp.concatenate([x, cen], axis=1)

    rj = lax.broadcasted_iota(jnp.int32, (N, N), 0)
    ci = lax.broadcasted_iota(jnp.int32, (N, N), 1)
    U = (ci > rj).astype(f32)

    g = x * fn
    agg32 = fn * jnp.dot(U, g, preferred_element_type=f32)

    cenT = cen.T
    parts = []
    for kk in range(3):
        cj = cen[:, kk:kk + 1]
        cirow = cenT[kk:kk + 1, :]
        w = jnp.abs(cj - cirow) * cirow * U
        parts.append(jnp.sum(w, axis=1, keepdims=True))
    agg3 = jnp.concatenate(parts, axis=1)

    agg = jnp.concatenate([agg32, agg3], axis=1)
    deg = (N - 1.0) - lax.broadcasted_iota(jnp.int32, (N, 1), 0).astype(f32)
    invdeg = 1.0 / jnp.maximum(deg, 1.0)
    hn1 = agg * invdeg
    h1 = (jnp.dot(h, w1s_ref[...], preferred_element_type=f32)
          + jnp.dot(hn1, w1n_ref[...], preferred_element_type=f32)
          + b1_ref[...][None, :])

    agg2 = jnp.dot(U, h1, preferred_element_type=f32)
    hn2 = agg2 * invdeg
    h2 = (jnp.dot(h1, w2s_ref[...], preferred_element_type=f32)
          + jnp.dot(hn2, w2n_ref[...], preferred_element_type=f32)
          + b2_ref[...][None, :])

    wm1 = wm1_ref[...]
    a_mat = jnp.dot(h2, wm1[:32, :], preferred_element_type=f32) + bm1_ref[...][None, :]
    b_mat = jnp.dot(h2, wm1[32:, :], preferred_element_type=f32)
    at_ref[:, :N] = a_mat.T
    at_ref[:, N:] = jnp.zeros((32, AT_PAD - N), f32)
    b_ref[...] = b_mat.T


def _stage2_body(at_hbm, b_hbm, wv_hbm, row0_hbm, col0_hbm,
                 cnt_hbm, bs_hbm, out_hbm,
                 at_v, b_v, wv_v, row0_v, col0_v, cnt_v, bs_v, out_v):
    i32 = jnp.int32
    wid = lax.axis_index("s") * NC + lax.axis_index("c")
    pltpu.sync_copy(at_hbm, at_v)
    pltpu.sync_copy(b_hbm, b_v)
    pltpu.sync_copy(wv_hbm, wv_v)
    pltpu.sync_copy(row0_hbm, row0_v)
    pltpu.sync_copy(col0_hbm, col0_v)
    pltpu.sync_copy(cnt_hbm, cnt_v)
    pltpu.sync_copy(bs_hbm, bs_v)

    def splat(v):
        return jnp.full((L,), v, i32)

    widv = splat(wid)
    i0 = jnp.max(plsc.load_gather(row0_v, [widv]))
    j0 = jnp.max(plsc.load_gather(col0_v, [widv]))
    cnt = jnp.max(plsc.load_gather(cnt_v, [widv]))
    bstart = jnp.max(plsc.load_gather(bs_v, [widv]))

    bm0 = plsc.load_gather(wv_v, [splat(2), splat(1)])
    bm1v = plsc.load_gather(wv_v, [splat(2), splat(2)])
    lane = lax.iota(i32, L)

    for half in range(2):
        ks = list(range(16 * half, 16 * half + 16))
        w0h = [plsc.load_gather(wv_v, [splat(0), splat(k + 1)]) for k in ks]
        w1h = [plsc.load_gather(wv_v, [splat(1), splat(k + 1)]) for k in ks]

        def row_body(state, half=half, ks=ks, w0h=w0h, w1h=w1h):
            i, jcur, ec = state
            seg = jnp.minimum(i - jcur, cnt - ec)
            iv = splat(i)
            bk = [plsc.load_gather(b_v, [splat(k), iv]) for k in ks]

            lead = jcur & (L - 1)
            base = jcur - lead
            nch = (lead + seg + (L - 1)) // L

            def ch_body(c, carry):
                off = base + c * L
                jj = off + lane
                m = (jj >= jcur) & (jj - jcur < seg)
                l = jnp.maximum(ec + jj - jcur, 0)
                idx0 = ((l >> 7) << 8) + (l & 127)
                if half == 0:
                    acc0 = bm0
                    acc1 = bm1v
                else:
                    acc0 = plsc.load_gather(out_v, [idx0])
                    acc1 = plsc.load_gather(out_v, [idx0 + 128])
                for kk, k in enumerate(ks):
                    a = at_v[k, pl.ds(off, L)]
                    t = jnp.maximum(a + bk[kk], 0.0)
                    acc0 = acc0 + t * w0h[kk]
                    acc1 = acc1 + t * w1h[kk]
                if half == 1:
                    acc0 = jnp.maximum(acc0, 0.0)
                    acc1 = jnp.maximum(acc1, 0.0)
                plsc.store_scatter(out_v, [idx0], acc0, mask=m)
                plsc.store_scatter(out_v, [idx0 + 128], acc1, mask=m)
                return carry

            lax.fori_loop(0, nch, ch_body, jnp.int32(0))
            jn = jcur + seg
            done_row = jn >= i
            return (jnp.where(done_row, i + 1, i),
                    jnp.where(done_row, 0, jn),
                    ec + seg)

        lax.while_loop(lambda s: s[2] < cnt, row_body,
                       (i0, j0, jnp.int32(0)))
    base = bstart * 256
    nfull = cnt >> 11

    def dma_body(c, carry):
        pltpu.sync_copy(out_v.at[pl.ds(c * 4096, 4096)],
                        out_hbm.at[pl.ds(base + c * 4096, 4096)])
        return carry

    lax.fori_loop(0, nfull, dma_body, jnp.int32(0))

    @pl.when((cnt & 2047) != 0)
    def _():
        pltpu.sync_copy(out_v.at[pl.ds(nfull * 4096, 3072)],
                        out_hbm.at[pl.ds(base + nfull * 4096, 3072)])


@jax.jit
def kernel(x, centroids, W1_self, W1_neigh, b1, W2_self, W2_neigh, b2,
           Wm1, bm1, Wm2, bm2):
    f32 = jnp.float32
    at, b_mat = pl.pallas_call(
        _stage1_body,
        out_shape=[jax.ShapeDtypeStruct((32, AT_PAD), f32),
                   jax.ShapeDtypeStruct((32, N), f32)],
    )(x, centroids, W1_self, W1_neigh, b1, W2_self, W2_neigh, b2, Wm1, bm1)

    z1 = jnp.zeros((1,), f32)
    wv = jnp.stack([
        jnp.concatenate([z1, Wm2[:, 0], z1]),
        jnp.concatenate([z1, Wm2[:, 1], z1]),
        jnp.concatenate([z1, bm2, jnp.zeros((31,), f32)]),
        jnp.zeros((34,), f32),
    ])
    mesh = plsc.VectorSubcoreMesh(core_axis_name="c", subcore_axis_name="s")
    stage2 = functools.partial(
        pl.kernel,
        out_type=jax.ShapeDtypeStruct((2 * E,), f32),
        mesh=mesh,
        compiler_params=pltpu.CompilerParams(needs_layout_passes=False),
        scratch_types=[
            pltpu.VMEM((32, AT_PAD), f32),
            pltpu.VMEM((32, N), f32),
            pltpu.VMEM((4, 34), f32),
            pltpu.VMEM((NW,), jnp.int32),
            pltpu.VMEM((NW,), jnp.int32),
            pltpu.VMEM((NW,), jnp.int32),
            pltpu.VMEM((NW,), jnp.int32),
            pltpu.VMEM((MAX_BLK * 256,), f32),
        ],
    )(_stage2_body)
    flat = stage2(at, b_mat, wv, jnp.asarray(_ROW0), jnp.asarray(_COL0),
                  jnp.asarray(_CNT), jnp.asarray(_BS))
    return flat.reshape(NBLOCKS, 2, 128).transpose(0, 2, 1).reshape(E, 2)


# speedup vs baseline: 1.2858x; 1.0474x over previous
import functools

import jax
import jax.numpy as jnp
import numpy as np
from jax import lax
from jax.experimental import pallas as pl
from jax.experimental.pallas import tpu as pltpu
from jax.experimental.pallas import tpu_sc as plsc

N = 1024
E = N * (N - 1) // 2
NC, NS, L = 2, 16, 16
NW = NC * NS
E_PER_W = E // NW
SPAN = 2 * E_PER_W
AT_PAD = 1056

NBLOCKS = E // 128
_ROW_COST = 30
_tri = (np.arange(N, dtype=np.int64) * (np.arange(N, dtype=np.int64) - 1)) // 2
_UNIT = 2048
_u_end = np.minimum(np.arange(1, (E + _UNIT - 1) // _UNIT + 1) * _UNIT, E)
_cost = _u_end + _ROW_COST * np.searchsorted(_tri, _u_end, side="left")
_targets = _cost[-1] * (np.arange(1, NW + 1) / NW)
_cuts = np.searchsorted(_cost, _targets, side="left")
_cuts[-1] = len(_u_end) - 1
_cuts = np.maximum.accumulate(np.minimum(_cuts, len(_u_end) - 1))
for _w in range(1, NW):
    if _cuts[_w] <= _cuts[_w - 1]:
        _cuts[_w] = _cuts[_w - 1] + 1
_ends = _u_end[_cuts]
_starts = np.concatenate([[0], _ends[:-1]])
_CNT = (_ends - _starts).astype(np.int32)
_BS = (_starts // 128).astype(np.int32)
MAX_BLK = int(np.max((_CNT + 127) // 128))
_ROW0 = (np.searchsorted(_tri, _starts, side="right") - 1).astype(np.int32)
_COL0 = (_starts - _tri[_ROW0]).astype(np.int32)


def _stage1_body(x_ref, cen_ref, w1s_ref, w1n_ref, b1_ref, w2s_ref, w2n_ref,
                 b2_ref, wm1_ref, bm1_ref, at_ref, b_ref):
    f32 = jnp.float32
    x = x_ref[...]
    cen = cen_ref[...]
    nrm = jnp.sqrt(jnp.sum(x * x, axis=1, keepdims=True))
    fn = x / jnp.maximum(nrm, 1e-12)
    h = jnp.concatenate([x, cen], axis=1)

    rj = lax.broadcasted_iota(jnp.int32, (N, N), 0)
    ci = lax.broadcasted_iota(jnp.int32, (N, N), 1)
    U = (ci > rj).astype(f32)

    g = x * fn
    agg32 = fn * jnp.dot(U, g, preferred_element_type=f32)

    cenT = cen.T
    parts = []
    for kk in range(3):
        cj = cen[:, kk:kk + 1]
        cirow = cenT[kk:kk + 1, :]
        w = jnp.abs(cj - cirow) * cirow * U
        parts.append(jnp.sum(w, axis=1, keepdims=True))
    agg3 = jnp.concatenate(parts, axis=1)

    agg = jnp.concatenate([agg32, agg3], axis=1)
    deg = (N - 1.0) - lax.broadcasted_iota(jnp.int32, (N, 1), 0).astype(f32)
    invdeg = 1.0 / jnp.maximum(deg, 1.0)
    hn1 = agg * invdeg
    h1 = (jnp.dot(h, w1s_ref[...], preferred_element_type=f32)
          + jnp.dot(hn1, w1n_ref[...], preferred_element_type=f32)
          + b1_ref[...][None, :])

    agg2 = jnp.dot(U, h1, preferred_element_type=f32)
    hn2 = agg2 * invdeg
    h2 = (jnp.dot(h1, w2s_ref[...], preferred_element_type=f32)
          + jnp.dot(hn2, w2n_ref[...], preferred_element_type=f32)
          + b2_ref[...][None, :])

    wm1 = wm1_ref[...]
    a_mat = jnp.dot(h2, wm1[:32, :], preferred_element_type=f32) + bm1_ref[...][None, :]
    b_mat = jnp.dot(h2, wm1[32:, :], preferred_element_type=f32)
    at_ref[:, :N] = a_mat.T
    at_ref[:, N:] = jnp.zeros((32, AT_PAD - N), f32)
    b_ref[...] = b_mat.T


def _stage2_body(at_hbm, b_hbm, wv_hbm, row0_hbm, col0_hbm,
                 cnt_hbm, bs_hbm, out_hbm,
                 at_v, b_v, wv_v, row0_v, col0_v, cnt_v, bs_v, out_v, sem):
    i32 = jnp.int32
    wid = lax.axis_index("s") * NC + lax.axis_index("c")
    descs = [pltpu.async_copy(src, dst, sem)
             for src, dst in ((at_hbm, at_v), (b_hbm, b_v), (wv_hbm, wv_v),
                              (row0_hbm, row0_v), (col0_hbm, col0_v),
                              (cnt_hbm, cnt_v), (bs_hbm, bs_v))]
    for d in descs:
        d.wait()

    def splat(v):
        return jnp.full((L,), v, i32)

    widv = splat(wid)
    i0 = jnp.max(plsc.load_gather(row0_v, [widv]))
    j0 = jnp.max(plsc.load_gather(col0_v, [widv]))
    cnt = jnp.max(plsc.load_gather(cnt_v, [widv]))
    bstart = jnp.max(plsc.load_gather(bs_v, [widv]))

    bm0 = plsc.load_gather(wv_v, [splat(2), splat(1)])
    bm1v = plsc.load_gather(wv_v, [splat(2), splat(2)])
    lane = lax.iota(i32, L)

    for half in range(2):
        ks = list(range(16 * half, 16 * half + 16))
        w0h = [plsc.load_gather(wv_v, [splat(0), splat(k + 1)]) for k in ks]
        w1h = [plsc.load_gather(wv_v, [splat(1), splat(k + 1)]) for k in ks]

        def row_body(state, half=half, ks=ks, w0h=w0h, w1h=w1h):
            i, jcur, ec = state
            seg = jnp.minimum(i - jcur, cnt - ec)
            iv = splat(i)
            bk = [plsc.load_gather(b_v, [splat(k), iv]) for k in ks]

            lead = jcur & (L - 1)
            base = jcur - lead
            nch = (lead + seg + (L - 1)) // L

            def ch_body(c, carry):
                off = base + c * L
                jj = off + lane
                m = (jj >= jcur) & (jj - jcur < seg)
                l = jnp.maximum(ec + jj - jcur, 0)
                idx0 = ((l >> 7) << 8) + (l & 127)
                if half == 0:
                    acc0 = bm0
                    acc1 = bm1v
                else:
                    acc0 = plsc.load_gather(out_v, [idx0])
                    acc1 = plsc.load_gather(out_v, [idx0 + 128])
                for kk, k in enumerate(ks):
                    a = at_v[k, pl.ds(off, L)]
                    t = jnp.maximum(a + bk[kk], 0.0)
                    acc0 = acc0 + t * w0h[kk]
                    acc1 = acc1 + t * w1h[kk]
                if half == 1:
                    acc0 = jnp.maximum(acc0, 0.0)
                    acc1 = jnp.maximum(acc1, 0.0)
                plsc.store_scatter(out_v, [idx0], acc0, mask=m)
                plsc.store_scatter(out_v, [idx0 + 128], acc1, mask=m)
                return carry

            lax.fori_loop(0, nch, ch_body, jnp.int32(0))
            jn = jcur + seg
            done_row = jn >= i
            return (jnp.where(done_row, i + 1, i),
                    jnp.where(done_row, 0, jn),
                    ec + seg)

        lax.while_loop(lambda s: s[2] < cnt, row_body,
                       (i0, j0, jnp.int32(0)))
    base = bstart * 256
    nfull = cnt >> 11

    def dma_fire(c, carry):
        pltpu.async_copy(out_v.at[pl.ds(c * 4096, 4096)],
                         out_hbm.at[pl.ds(base + c * 4096, 4096)], sem)
        return carry

    lax.fori_loop(0, nfull, dma_fire, jnp.int32(0))

    @pl.when((cnt & 2047) != 0)
    def _():
        pltpu.async_copy(out_v.at[pl.ds(nfull * 4096, 3072)],
                         out_hbm.at[pl.ds(base + nfull * 4096, 3072)], sem)

    def dma_drain(c, carry):
        pltpu.make_async_copy(out_v.at[pl.ds(c * 4096, 4096)],
                              out_hbm.at[pl.ds(base + c * 4096, 4096)],
                              sem).wait()
        return carry

    lax.fori_loop(0, nfull, dma_drain, jnp.int32(0))

    @pl.when((cnt & 2047) != 0)
    def _():
        pltpu.make_async_copy(out_v.at[pl.ds(nfull * 4096, 3072)],
                              out_hbm.at[pl.ds(base + nfull * 4096, 3072)],
                              sem).wait()


@jax.jit
def kernel(x, centroids, W1_self, W1_neigh, b1, W2_self, W2_neigh, b2,
           Wm1, bm1, Wm2, bm2):
    f32 = jnp.float32
    at, b_mat = pl.pallas_call(
        _stage1_body,
        out_shape=[jax.ShapeDtypeStruct((32, AT_PAD), f32),
                   jax.ShapeDtypeStruct((32, N), f32)],
    )(x, centroids, W1_self, W1_neigh, b1, W2_self, W2_neigh, b2, Wm1, bm1)

    z1 = jnp.zeros((1,), f32)
    wv = jnp.stack([
        jnp.concatenate([z1, Wm2[:, 0], z1]),
        jnp.concatenate([z1, Wm2[:, 1], z1]),
        jnp.concatenate([z1, bm2, jnp.zeros((31,), f32)]),
        jnp.zeros((34,), f32),
    ])
    mesh = plsc.VectorSubcoreMesh(core_axis_name="c", subcore_axis_name="s")
    stage2 = functools.partial(
        pl.kernel,
        out_type=jax.ShapeDtypeStruct((2 * E,), f32),
        mesh=mesh,
        compiler_params=pltpu.CompilerParams(needs_layout_passes=False),
        scratch_types=[
            pltpu.VMEM((32, AT_PAD), f32),
            pltpu.VMEM((32, N), f32),
            pltpu.VMEM((4, 34), f32),
            pltpu.VMEM((NW,), jnp.int32),
            pltpu.VMEM((NW,), jnp.int32),
            pltpu.VMEM((NW,), jnp.int32),
            pltpu.VMEM((NW,), jnp.int32),
            pltpu.VMEM((MAX_BLK * 256,), f32),
            pltpu.SemaphoreType.DMA,
        ],
    )(_stage2_body)
    flat = stage2(at, b_mat, wv, jnp.asarray(_ROW0), jnp.asarray(_COL0),
                  jnp.asarray(_CNT), jnp.asarray(_BS))
    return flat.reshape(NBLOCKS, 2, 128).transpose(0, 2, 1).reshape(E, 2)
